# baseline (speedup 1.0000x reference)
import functools

import jax
import jax.numpy as jnp
from jax import lax
from jax.experimental import pallas as pl
from jax.experimental.pallas import tpu as pltpu
from jax.experimental.pallas import tpu_sc as plsc

B, N, D = 64, 8192, 64
ROWS = B * N

NC, NS = 2, 16
NW = NC * NS
ROWS_PER_W = ROWS // NW
CHUNK = 1024
NCHUNK = ROWS_PER_W // CHUNK


def _z_kernel(x_ref, p_ref, z_ref):
    z_ref[0, 0, :] = jnp.dot(x_ref[0], p_ref[...])[:, 0]


def _compute_z(x, projection):
    z3 = pl.pallas_call(
        _z_kernel,
        grid=(B,),
        in_specs=[
            pl.BlockSpec((1, N, D), lambda b: (b, 0, 0)),
            pl.BlockSpec((D, 1), lambda b: (0, 0)),
        ],
        out_specs=pl.BlockSpec((1, 1, N), lambda b: (b, 0, 0)),
        out_shape=jax.ShapeDtypeStruct((B, 1, N), jnp.float32),
    )(x, projection)
    return z3[:, 0, :]


def _sc_gather_body(x_hbm, idx_hbm, out_hbm, idx_v, rows_v, sem):
    wid = lax.axis_index("s") * NC + lax.axis_index("c")
    row_base = wid * ROWS_PER_W

    def chunk_body(c, _):
        base = row_base + c * CHUNK
        irow = pl.multiple_of(base // 128, 8)
        pltpu.sync_copy(idx_hbm.at[pl.ds(irow, CHUNK // 128), :], idx_v)
        copies = []
        for j in range(CHUNK // 128):
            copies.append(
                pltpu.async_copy(
                    x_hbm.at[idx_v.at[j]],
                    rows_v.at[pl.ds(j * 128, 128), :],
                    sem,
                )
            )
        for cp in copies:
            cp.wait()
        pltpu.sync_copy(rows_v, out_hbm.at[pl.ds(base, CHUNK), :])
        return _

    lax.fori_loop(0, NCHUNK, chunk_body, None)


@functools.partial(jax.jit, static_argnums=())
def _sc_gather(x_flat, idx2d):
    mesh = plsc.VectorSubcoreMesh(
        core_axis_name="c", subcore_axis_name="s", num_cores=NC, num_subcores=NS
    )
    kern = functools.partial(
        pl.kernel,
        out_type=jax.ShapeDtypeStruct((ROWS, D), jnp.float32),
        mesh=mesh,
        compiler_params=pltpu.CompilerParams(use_tc_tiling_on_sc=False),
        scratch_types=[
            pltpu.VMEM((CHUNK // 128, 128), jnp.int32),
            pltpu.VMEM((CHUNK, D), jnp.float32),
            pltpu.SemaphoreType.DMA,
        ],
    )(_sc_gather_body)
    return kern(x_flat, idx2d)


def kernel(x, projection):
    z = _compute_z(x, projection)
    idx = jnp.argsort(z, axis=1).astype(jnp.int32)
    gidx = idx + (jnp.arange(B, dtype=jnp.int32) * N)[:, None]
    out_flat = _sc_gather(x.reshape(ROWS, D), gidx.reshape(ROWS // 128, 128))
    return out_flat.reshape(B, N, D)


# speedup vs baseline: 1.0501x; 1.0501x over previous
import functools

import jax
import jax.numpy as jnp
from jax import lax
from jax.experimental import pallas as pl
from jax.experimental.pallas import tpu as pltpu
from jax.experimental.pallas import tpu_sc as plsc

B, N, D = 64, 8192, 64
ROWS = B * N

NC, NS = 2, 16
NW = NC * NS
ROWS_PER_W = ROWS // NW
CHUNK = 1024
NCHUNK = ROWS_PER_W // CHUNK


def _z_kernel(x_ref, p_ref, z_ref):
    z_ref[0, 0, :] = jnp.dot(x_ref[0], p_ref[...])[:, 0]


def _compute_z(x, projection):
    z3 = pl.pallas_call(
        _z_kernel,
        grid=(B,),
        in_specs=[
            pl.BlockSpec((1, N, D), lambda b: (b, 0, 0)),
            pl.BlockSpec((D, 1), lambda b: (0, 0)),
        ],
        out_specs=pl.BlockSpec((1, 1, N), lambda b: (b, 0, 0)),
        out_shape=jax.ShapeDtypeStruct((B, 1, N), jnp.float32),
    )(x, projection)
    return z3[:, 0, :]


R, L = N // 128, 128
NBITS = 13


def _sort_kernel(z_ref, out_ref):
    key = z_ref[...]
    row_io = lax.broadcasted_iota(jnp.int32, (B, R, L), 1)
    lane_io = lax.broadcasted_iota(jnp.int32, (B, R, L), 2)
    I = row_io * L + lane_io
    idx = I
    for k in range(1, NBITS + 1):
        for j in range(k - 1, -1, -1):
            d = 1 << j
            if d < L:
                pk1 = pltpu.roll(key, L - d, 2)
                pk2 = pltpu.roll(key, d, 2)
                pi1 = pltpu.roll(idx, L - d, 2)
                pi2 = pltpu.roll(idx, d, 2)
            else:
                dr = d // L
                pk1 = pltpu.roll(key, R - dr, 1)
                pk2 = pltpu.roll(key, dr, 1)
                pi1 = pltpu.roll(idx, R - dr, 1)
                pi2 = pltpu.roll(idx, dr, 1)
            is_lo = (I & d) == 0
            pkey = jnp.where(is_lo, pk1, pk2)
            pidx = jnp.where(is_lo, pi1, pi2)
            up = (I & (1 << k)) == 0
            less_self = (key < pkey) | ((key == pkey) & (idx < pidx))
            choose_self = less_self == (up == is_lo)
            key = jnp.where(choose_self, key, pkey)
            idx = jnp.where(choose_self, idx, pidx)
    b_io = lax.broadcasted_iota(jnp.int32, (B, R, L), 0)
    out_ref[...] = idx + b_io * N


def _argsort(z):
    return pl.pallas_call(
        _sort_kernel,
        out_shape=jax.ShapeDtypeStruct((B, R, L), jnp.int32),
    )(z.reshape(B, R, L))


def _sc_gather_body(x_hbm, idx_hbm, out_hbm, idx_v, rows_v, sem):
    wid = lax.axis_index("s") * NC + lax.axis_index("c")
    row_base = wid * ROWS_PER_W

    def chunk_body(c, _):
        base = row_base + c * CHUNK
        irow = pl.multiple_of(base // 128, 8)
        pltpu.sync_copy(idx_hbm.at[pl.ds(irow, CHUNK // 128), :], idx_v)
        copies = []
        for j in range(CHUNK // 128):
            copies.append(
                pltpu.async_copy(
                    x_hbm.at[idx_v.at[j]],
                    rows_v.at[pl.ds(j * 128, 128), :],
                    sem,
                )
            )
        for cp in copies:
            cp.wait()
        pltpu.sync_copy(rows_v, out_hbm.at[pl.ds(base, CHUNK), :])
        return _

    lax.fori_loop(0, NCHUNK, chunk_body, None)


@functools.partial(jax.jit, static_argnums=())
def _sc_gather(x_flat, idx2d):
    mesh = plsc.VectorSubcoreMesh(
        core_axis_name="c", subcore_axis_name="s", num_cores=NC, num_subcores=NS
    )
    kern = functools.partial(
        pl.kernel,
        out_type=jax.ShapeDtypeStruct((ROWS, D), jnp.float32),
        mesh=mesh,
        compiler_params=pltpu.CompilerParams(use_tc_tiling_on_sc=False),
        scratch_types=[
            pltpu.VMEM((CHUNK // 128, 128), jnp.int32),
            pltpu.VMEM((CHUNK, D), jnp.float32),
            pltpu.SemaphoreType.DMA,
        ],
    )(_sc_gather_body)
    return kern(x_flat, idx2d)


def kernel(x, projection):
    z = _compute_z(x, projection)
    gidx = _argsort(z)
    out_flat = _sc_gather(x.reshape(ROWS, D), gidx.reshape(ROWS // 128, 128))
    return out_flat.reshape(B, N, D)


# speedup vs baseline: 1.2240x; 1.1656x over previous
import functools

import jax
import jax.numpy as jnp
from jax import lax
from jax.experimental import pallas as pl
from jax.experimental.pallas import tpu as pltpu
from jax.experimental.pallas import tpu_sc as plsc

B, N, D = 64, 8192, 64
ROWS = B * N

NC, NS = 2, 16
NW = NC * NS
ROWS_PER_W = ROWS // NW
CHUNK = 1024
NCHUNK = ROWS_PER_W // CHUNK


def _z_kernel(x_ref, p_ref, z_ref):
    z_ref[0, 0, :] = jnp.dot(x_ref[0], p_ref[...])[:, 0]


def _compute_z(x, projection):
    z3 = pl.pallas_call(
        _z_kernel,
        grid=(B,),
        in_specs=[
            pl.BlockSpec((1, N, D), lambda b: (b, 0, 0)),
            pl.BlockSpec((D, 1), lambda b: (0, 0)),
        ],
        out_specs=pl.BlockSpec((1, 1, N), lambda b: (b, 0, 0)),
        out_shape=jax.ShapeDtypeStruct((B, 1, N), jnp.float32),
    )(x, projection)
    return z3[:, 0, :]


R, L = N // 128, 128
NBITS = 13


def _sort_kernel(z_ref, out_ref):
    key = z_ref[...]
    row_io = lax.broadcasted_iota(jnp.int32, (B, R, L), 1)
    lane_io = lax.broadcasted_iota(jnp.int32, (B, R, L), 2)
    I = row_io * L + lane_io
    idx = I
    for k in range(1, NBITS + 1):
        for j in range(k - 1, -1, -1):
            d = 1 << j
            if d < L:
                pk1 = pltpu.roll(key, L - d, 2)
                pk2 = pltpu.roll(key, d, 2)
                pi1 = pltpu.roll(idx, L - d, 2)
                pi2 = pltpu.roll(idx, d, 2)
            else:
                dr = d // L
                pk1 = pltpu.roll(key, R - dr, 1)
                pk2 = pltpu.roll(key, dr, 1)
                pi1 = pltpu.roll(idx, R - dr, 1)
                pi2 = pltpu.roll(idx, dr, 1)
            is_lo = (I & d) == 0
            pkey = jnp.where(is_lo, pk1, pk2)
            pidx = jnp.where(is_lo, pi1, pi2)
            up = (I & (1 << k)) == 0
            less_self = (key < pkey) | ((key == pkey) & (idx < pidx))
            choose_self = less_self == (up == is_lo)
            key = jnp.where(choose_self, key, pkey)
            idx = jnp.where(choose_self, idx, pidx)
    b_io = lax.broadcasted_iota(jnp.int32, (B, R, L), 0)
    out_ref[...] = idx + b_io * N


def _argsort(z):
    return pl.pallas_call(
        _sort_kernel,
        out_shape=jax.ShapeDtypeStruct((B, R, L), jnp.int32),
    )(z.reshape(B, R, L))


BATCHES_PER_TILE = B // NW


def _sc_sort_body(z_hbm, idx_hbm, stage_f, ku_a, id_a, ku_b, id_b, hist):
    wid = lax.axis_index("s") * NC + lax.axis_index("c")
    lane = lax.iota(jnp.int32, 16)
    ones = jnp.ones((16,), jnp.int32)
    nvec = N // 16
    blk = N // 16

    for lb in range(BATCHES_PER_TILE):
        b = wid * BATCHES_PER_TILE + lb
        pltpu.sync_copy(z_hbm.at[b], stage_f)

        def conv(i, c):
            v = stage_f[pl.ds(i * 16, 16)]
            u = lax.bitcast_convert_type(v, jnp.int32)
            m = lax.shift_right_arithmetic(u, 31) | jnp.int32(-2147483648)
            ku_a[pl.ds(i * 16, 16)] = u ^ m
            id_a[pl.ds(i * 16, 16)] = i * 16 + lane
            return c

        lax.fori_loop(0, nvec, conv, 0)

        bufs = [(ku_a, id_a), (ku_b, id_b)]
        for p in range(4):
            src_ku, src_id = bufs[p % 2]
            dst_ku, dst_id = bufs[(p + 1) % 2]
            sh = 8 * p
            last = p == 3

            def zero(i, c):
                hist[pl.ds(i * 16, 16)] = jnp.zeros((16,), jnp.int32)
                return c

            lax.fori_loop(0, 256, zero, 0)

            def histo(i, idxv):
                k = plsc.load_gather(src_ku, [idxv])
                digit = lax.shift_right_logical(k, sh) & 0xFF
                binv = digit * 16 + lane
                plsc.addupdate_scatter(hist, [binv], ones)
                return idxv + 1

            lax.fori_loop(0, blk, histo, lane * blk)

            def scan(i, carry):
                v = hist[pl.ds(i * 16, 16)]
                cum = plsc.cumsum(v)
                hist[pl.ds(i * 16, 16)] = carry + (cum - v)
                return carry + jnp.sum(v)

            lax.fori_loop(0, 256, scan, jnp.int32(0))

            def permute(i, idxv):
                k = plsc.load_gather(src_ku, [idxv])
                v = plsc.load_gather(src_id, [idxv])
                digit = lax.shift_right_logical(k, sh) & 0xFF
                binv = digit * 16 + lane
                off = plsc.load_gather(hist, [binv])
                if last:
                    plsc.store_scatter(dst_id, [off], v + b * N)
                else:
                    plsc.store_scatter(dst_ku, [off], k)
                    plsc.store_scatter(dst_id, [off], v)
                plsc.addupdate_scatter(hist, [binv], ones)
                return idxv + 1

            lax.fori_loop(0, blk, permute, lane * blk)

        pltpu.sync_copy(id_a, idx_hbm.at[b])


def _sc_argsort(z):
    mesh = plsc.VectorSubcoreMesh(
        core_axis_name="c", subcore_axis_name="s", num_cores=NC, num_subcores=NS
    )
    kern = functools.partial(
        pl.kernel,
        out_type=jax.ShapeDtypeStruct((B, N), jnp.int32),
        mesh=mesh,
        compiler_params=pltpu.CompilerParams(
            use_tc_tiling_on_sc=False, needs_layout_passes=False
        ),
        scratch_types=[
            pltpu.VMEM((N,), jnp.float32),
            pltpu.VMEM((N,), jnp.int32),
            pltpu.VMEM((N,), jnp.int32),
            pltpu.VMEM((N,), jnp.int32),
            pltpu.VMEM((N,), jnp.int32),
            pltpu.VMEM((4096,), jnp.int32),
        ],
    )(_sc_sort_body)
    return kern(z)


def _sc_gather_body(x_hbm, idx_hbm, out_hbm, idx_v, rows_v, sem):
    wid = lax.axis_index("s") * NC + lax.axis_index("c")
    row_base = wid * ROWS_PER_W

    def chunk_body(c, _):
        base = row_base + c * CHUNK
        irow = pl.multiple_of(base // 128, 8)
        pltpu.sync_copy(idx_hbm.at[pl.ds(irow, CHUNK // 128), :], idx_v)
        copies = []
        for j in range(CHUNK // 128):
            copies.append(
                pltpu.async_copy(
                    x_hbm.at[idx_v.at[j]],
                    rows_v.at[pl.ds(j * 128, 128), :],
                    sem,
                )
            )
        for cp in copies:
            cp.wait()
        pltpu.sync_copy(rows_v, out_hbm.at[pl.ds(base, CHUNK), :])
        return _

    lax.fori_loop(0, NCHUNK, chunk_body, None)


@functools.partial(jax.jit, static_argnums=())
def _sc_gather(x_flat, idx2d):
    mesh = plsc.VectorSubcoreMesh(
        core_axis_name="c", subcore_axis_name="s", num_cores=NC, num_subcores=NS
    )
    kern = functools.partial(
        pl.kernel,
        out_type=jax.ShapeDtypeStruct((ROWS, D), jnp.float32),
        mesh=mesh,
        compiler_params=pltpu.CompilerParams(use_tc_tiling_on_sc=False),
        scratch_types=[
            pltpu.VMEM((CHUNK // 128, 128), jnp.int32),
            pltpu.VMEM((CHUNK, D), jnp.float32),
            pltpu.SemaphoreType.DMA,
        ],
    )(_sc_gather_body)
    return kern(x_flat, idx2d)


def kernel(x, projection):
    z = _compute_z(x, projection)
    gidx = _sc_argsort(z)
    out_flat = _sc_gather(x.reshape(ROWS, D), gidx.reshape(ROWS // 128, 128))
    return out_flat.reshape(B, N, D)


# speedup vs baseline: 1.2296x; 1.0045x over previous
import functools

import jax
import jax.numpy as jnp
from jax import lax
from jax.experimental import pallas as pl
from jax.experimental.pallas import tpu as pltpu
from jax.experimental.pallas import tpu_sc as plsc

B, N, D = 64, 8192, 64
ROWS = B * N

NC, NS = 2, 16
NW = NC * NS
ROWS_PER_W = ROWS // NW
CHUNK = 1024
NCHUNK = ROWS_PER_W // CHUNK


def _z_kernel(x_ref, p_ref, z_ref):
    for i in range(4):
        z_ref[i, 0, :] = jnp.dot(x_ref[i], p_ref[...])[:, 0]


def _compute_z(x, projection):
    z3 = pl.pallas_call(
        _z_kernel,
        grid=(B // 4,),
        in_specs=[
            pl.BlockSpec((4, N, D), lambda b: (b, 0, 0)),
            pl.BlockSpec((D, 1), lambda b: (0, 0)),
        ],
        out_specs=pl.BlockSpec((4, 1, N), lambda b: (b, 0, 0)),
        out_shape=jax.ShapeDtypeStruct((B, 1, N), jnp.float32),
    )(x, projection)
    return z3[:, 0, :]


BATCHES_PER_TILE = B // NW


def _sc_sort_body(z_hbm, idx_hbm, stage_f, ku_a, id_a, ku_b, id_b, hist):
    wid = lax.axis_index("s") * NC + lax.axis_index("c")
    lane = lax.iota(jnp.int32, 16)
    ones = jnp.ones((16,), jnp.int32)
    nvec = N // 16
    blk = N // 16

    for lb in range(BATCHES_PER_TILE):
        b = wid * BATCHES_PER_TILE + lb
        pltpu.sync_copy(z_hbm.at[b], stage_f)

        def conv(i, c):
            v = stage_f[pl.ds(i * 16, 16)]
            u = lax.bitcast_convert_type(v, jnp.int32)
            m = lax.shift_right_arithmetic(u, 31) | jnp.int32(-2147483648)
            ku_a[pl.ds(i * 16, 16)] = u ^ m
            id_a[pl.ds(i * 16, 16)] = i * 16 + lane
            return c

        lax.fori_loop(0, nvec, conv, 0)

        bufs = [(ku_a, id_a), (ku_b, id_b)]
        for p in range(4):
            src_ku, src_id = bufs[p % 2]
            dst_ku, dst_id = bufs[(p + 1) % 2]
            sh = 8 * p
            last = p == 3

            def zero(i, c):
                hist[pl.ds(i * 16, 16)] = jnp.zeros((16,), jnp.int32)
                return c

            lax.fori_loop(0, 256, zero, 0)

            def histo(i, idxv):
                k = plsc.load_gather(src_ku, [idxv])
                digit = lax.shift_right_logical(k, sh) & 0xFF
                binv = digit * 16 + lane
                plsc.addupdate_scatter(hist, [binv], ones)
                return idxv + 1

            lax.fori_loop(0, blk, histo, lane * blk)

            def scan(i, carry):
                v = hist[pl.ds(i * 16, 16)]
                cum = plsc.cumsum(v)
                hist[pl.ds(i * 16, 16)] = carry + (cum - v)
                return carry + jnp.sum(v)

            lax.fori_loop(0, 256, scan, jnp.int32(0))

            def permute(i, idxv):
                k = plsc.load_gather(src_ku, [idxv])
                v = plsc.load_gather(src_id, [idxv])
                digit = lax.shift_right_logical(k, sh) & 0xFF
                binv = digit * 16 + lane
                off = plsc.load_gather(hist, [binv])
                if last:
                    plsc.store_scatter(dst_id, [off], v + b * N)
                else:
                    plsc.store_scatter(dst_ku, [off], k)
                    plsc.store_scatter(dst_id, [off], v)
                plsc.addupdate_scatter(hist, [binv], ones)
                return idxv + 1

            lax.fori_loop(0, blk, permute, lane * blk)

        pltpu.sync_copy(id_a, idx_hbm.at[b])


def _sc_argsort(z):
    mesh = plsc.VectorSubcoreMesh(
        core_axis_name="c", subcore_axis_name="s", num_cores=NC, num_subcores=NS
    )
    kern = functools.partial(
        pl.kernel,
        out_type=jax.ShapeDtypeStruct((B, N), jnp.int32),
        mesh=mesh,
        compiler_params=pltpu.CompilerParams(
            use_tc_tiling_on_sc=False, needs_layout_passes=False
        ),
        scratch_types=[
            pltpu.VMEM((N,), jnp.float32),
            pltpu.VMEM((N,), jnp.int32),
            pltpu.VMEM((N,), jnp.int32),
            pltpu.VMEM((N,), jnp.int32),
            pltpu.VMEM((N,), jnp.int32),
            pltpu.VMEM((4096,), jnp.int32),
        ],
    )(_sc_sort_body)
    return kern(z)


def _sc_gather_body(x_hbm, idx_hbm, out_hbm, idx_v, rows_v, sem):
    wid = lax.axis_index("s") * NC + lax.axis_index("c")
    row_base = wid * ROWS_PER_W

    def chunk_body(c, _):
        base = row_base + c * CHUNK
        irow = pl.multiple_of(base // 128, 8)
        pltpu.sync_copy(idx_hbm.at[pl.ds(irow, CHUNK // 128), :], idx_v)
        copies = []
        for j in range(CHUNK // 128):
            copies.append(
                pltpu.async_copy(
                    x_hbm.at[idx_v.at[j]],
                    rows_v.at[pl.ds(j * 128, 128), :],
                    sem,
                )
            )
        for cp in copies:
            cp.wait()
        pltpu.sync_copy(rows_v, out_hbm.at[pl.ds(base, CHUNK), :])
        return _

    lax.fori_loop(0, NCHUNK, chunk_body, None)


@functools.partial(jax.jit, static_argnums=())
def _sc_gather(x_flat, idx2d):
    mesh = plsc.VectorSubcoreMesh(
        core_axis_name="c", subcore_axis_name="s", num_cores=NC, num_subcores=NS
    )
    kern = functools.partial(
        pl.kernel,
        out_type=jax.ShapeDtypeStruct((ROWS, D), jnp.float32),
        mesh=mesh,
        compiler_params=pltpu.CompilerParams(use_tc_tiling_on_sc=False),
        scratch_types=[
            pltpu.VMEM((CHUNK // 128, 128), jnp.int32),
            pltpu.VMEM((CHUNK, D), jnp.float32),
            pltpu.SemaphoreType.DMA,
        ],
    )(_sc_gather_body)
    return kern(x_flat, idx2d)


def kernel(x, projection):
    z = _compute_z(x, projection)
    gidx = _sc_argsort(z)
    out_flat = _sc_gather(x.reshape(ROWS, D), gidx.reshape(ROWS // 128, 128))
    return out_flat.reshape(B, N, D)
